# Initial kernel scaffold; baseline (speedup 1.0000x reference)
import jax
import jax.numpy as jnp
from jax.experimental import pallas as pl


def kernel(sentence_ids, dataset_ids, cache):
    raise NotImplementedError("write your pallas kernel here")


# speedup vs baseline: 1.1456x; 1.1456x over previous
import functools

import jax
import jax.numpy as jnp
from jax import lax
from jax.experimental import pallas as pl
from jax.experimental.pallas import tpu as pltpu
from jax.experimental.pallas import tpu_sc as plsc

_NUM_SENTENCES = 100000
_SEQ = 20
_DIM = 64
_ROW = _SEQ * _DIM

_NC = 2
_NS = 16
_NW = _NC * _NS
_LANES = 16

_C = 64


def kernel(sentence_ids, dataset_ids, cache):
    batch = sentence_ids.shape[0]
    b_per_w = batch // _NW
    nchunk = b_per_w // _C
    cache2d = cache.reshape(cache.shape[0], _ROW)

    mesh = plsc.VectorSubcoreMesh(
        core_axis_name="c", subcore_axis_name="s",
        num_cores=_NC, num_subcores=_NS)

    @functools.partial(
        pl.kernel,
        mesh=mesh,
        out_type=jax.ShapeDtypeStruct((batch, _DIM), jnp.float32),
        scratch_types=[
            pltpu.VMEM((b_per_w,), jnp.int32),
            pltpu.VMEM((b_per_w,), jnp.int32),
            pltpu.VMEM((nchunk, _C), jnp.int32),
            pltpu.VMEM((_C, _ROW), jnp.float32),
            pltpu.VMEM((_C, _DIM), jnp.float32),
            pltpu.SemaphoreType.DMA,
        ],
    )
    def sc_kernel(sid_hbm, did_hbm, cache_hbm, out_hbm,
                  sid_v, did_v, idx_v, rows_v, out_v, sem):
        wid = lax.axis_index("s") * _NC + lax.axis_index("c")
        base = wid * b_per_w

        pltpu.sync_copy(sid_hbm.at[pl.ds(base, b_per_w)], sid_v)
        pltpu.sync_copy(did_hbm.at[pl.ds(base, b_per_w)], did_v)

        per_chunk = _C // _LANES
        for g in range(nchunk):
            for h in range(per_chunk):
                j = (g * per_chunk + h) * _LANES
                v = (sid_v[pl.ds(j, _LANES)]
                     + did_v[pl.ds(j, _LANES)] * _NUM_SENTENCES)
                idx_v[g, pl.ds(h * _LANES, _LANES)] = v

        def chunk_body(g, carry):
            pltpu.async_copy(cache_hbm.at[idx_v.at[g]], rows_v, sem).wait()

            def pool_body(s, carry2):
                for d in range(_DIM // _LANES):
                    acc = rows_v[s, pl.ds(d * _LANES, _LANES)]
                    for t in range(1, _SEQ):
                        acc = acc + rows_v[s, pl.ds(t * _DIM + d * _LANES,
                                                    _LANES)]
                    out_v[s, pl.ds(d * _LANES, _LANES)] = acc * (1.0 / _SEQ)
                return carry2

            lax.fori_loop(0, _C, pool_body, 0, unroll=False)
            pltpu.sync_copy(out_v, out_hbm.at[pl.ds(base + g * _C, _C)])
            return carry

        lax.fori_loop(0, nchunk, chunk_body, 0, unroll=False)

    return sc_kernel(sentence_ids, dataset_ids, cache2d)
